# baseline (speedup 1.0000x reference)
import functools

import jax
import jax.numpy as jnp
from jax import lax
from jax.experimental import pallas as pl

N_TOKENS = 32768
EMB = 768
NE = 8
NSPLIT = 1
Q = 8192
STEP = NSPLIT * Q


def _router_quarter(x, w, j, logitsT_ref, probsT_ref, idxT_ref, wtsT_ref):
    lgT = lax.dot_general(w, x, (((0,), (1,)), ((), ())),
                          preferred_element_type=jnp.float32)
    logitsT_ref[:, pl.ds(j * Q, Q)] = lgT
    mT = jnp.max(lgT, axis=0, keepdims=True)
    eT = jnp.exp(lgT - mT)
    pT = eT / jnp.sum(eT, axis=0, keepdims=True)
    probsT_ref[:, pl.ds(j * Q, Q)] = pT

    iota = lax.broadcasted_iota(jnp.int32, (NE, Q), 0)
    p1 = jnp.max(pT, axis=0, keepdims=True)
    i1 = jnp.min(jnp.where(pT == p1, iota, NE), axis=0, keepdims=True)
    masked = jnp.where(iota == i1, -jnp.inf, pT)
    p2 = jnp.max(masked, axis=0, keepdims=True)
    i2 = jnp.min(jnp.where(masked == p2, iota, NE), axis=0, keepdims=True)
    denom = jnp.maximum(p1 + p2, 1e-8)
    idxT_ref[:, pl.ds(j * Q, Q)] = jnp.concatenate([i1, i2], axis=0)
    wtsT_ref[:, pl.ds(j * Q, Q)] = jnp.concatenate(
        [p1 / denom, p2 / denom], axis=0)


def _router_block(*refs):
    x_refs = refs[:NSPLIT]
    w_ref = refs[NSPLIT]
    logitsT_ref, probsT_ref, idxT_ref, wtsT_ref = refs[NSPLIT + 1:]
    w = w_ref[...]
    for j in range(NSPLIT):
        _router_quarter(x_refs[j][...], w, j,
                        logitsT_ref, probsT_ref, idxT_ref, wtsT_ref)


@jax.jit
def kernel(noise_clock_emb, route_weight):
    grid = N_TOKENS // STEP

    def band(j):
        return pl.BlockSpec((Q, EMB), lambda i, j=j: (NSPLIT * i + j, 0))

    in_specs = [band(j) for j in range(NSPLIT)]
    in_specs.append(pl.BlockSpec((EMB, NE), lambda i: (0, 0)))
    out_specs = [
        pl.BlockSpec((NE, STEP), lambda i: (0, i)),
        pl.BlockSpec((NE, STEP), lambda i: (0, i)),
        pl.BlockSpec((2, STEP), lambda i: (0, i)),
        pl.BlockSpec((2, STEP), lambda i: (0, i)),
    ]
    out_shape = (
        jax.ShapeDtypeStruct((NE, N_TOKENS), jnp.float32),
        jax.ShapeDtypeStruct((NE, N_TOKENS), jnp.float32),
        jax.ShapeDtypeStruct((2, N_TOKENS), jnp.int32),
        jax.ShapeDtypeStruct((2, N_TOKENS), jnp.float32),
    )
    logitsT, probsT, idxT, wtsT = pl.pallas_call(
        _router_block,
        grid=(grid,),
        in_specs=in_specs,
        out_specs=out_specs,
        out_shape=out_shape,
    )(*([noise_clock_emb] * NSPLIT), route_weight)
    return (logitsT.T, probsT.T, idxT.T, wtsT.T)


# speedup vs baseline: 1.0594x; 1.0594x over previous
import functools

import jax
import jax.numpy as jnp
from jax import lax
from jax.experimental import pallas as pl

N_TOKENS = 32768
EMB = 768
NE = 8
NSPLIT = 1
Q = 2048
STEP = NSPLIT * Q


def _router_quarter(x, w, j, logitsT_ref, probsT_ref, idxT_ref, wtsT_ref):
    lgT = lax.dot_general(w, x, (((0,), (1,)), ((), ())),
                          preferred_element_type=jnp.float32)
    logitsT_ref[:, pl.ds(j * Q, Q)] = lgT
    mT = jnp.max(lgT, axis=0, keepdims=True)
    eT = jnp.exp(lgT - mT)
    pT = eT / jnp.sum(eT, axis=0, keepdims=True)
    probsT_ref[:, pl.ds(j * Q, Q)] = pT

    iota = lax.broadcasted_iota(jnp.int32, (NE, Q), 0)
    p1 = jnp.max(pT, axis=0, keepdims=True)
    i1 = jnp.min(jnp.where(pT == p1, iota, NE), axis=0, keepdims=True)
    masked = jnp.where(iota == i1, -jnp.inf, pT)
    p2 = jnp.max(masked, axis=0, keepdims=True)
    i2 = jnp.min(jnp.where(masked == p2, iota, NE), axis=0, keepdims=True)
    denom = jnp.maximum(p1 + p2, 1e-8)
    idxT_ref[:, pl.ds(j * Q, Q)] = jnp.concatenate([i1, i2], axis=0)
    wtsT_ref[:, pl.ds(j * Q, Q)] = jnp.concatenate(
        [p1 / denom, p2 / denom], axis=0)


def _router_block(*refs):
    x_refs = refs[:NSPLIT]
    w_ref = refs[NSPLIT]
    logitsT_ref, probsT_ref, idxT_ref, wtsT_ref = refs[NSPLIT + 1:]
    w = w_ref[...]
    for j in range(NSPLIT):
        _router_quarter(x_refs[j][...], w, j,
                        logitsT_ref, probsT_ref, idxT_ref, wtsT_ref)


@jax.jit
def kernel(noise_clock_emb, route_weight):
    grid = N_TOKENS // STEP

    def band(j):
        return pl.BlockSpec((Q, EMB), lambda i, j=j: (NSPLIT * i + j, 0))

    in_specs = [band(j) for j in range(NSPLIT)]
    in_specs.append(pl.BlockSpec((EMB, NE), lambda i: (0, 0)))
    out_specs = [
        pl.BlockSpec((NE, STEP), lambda i: (0, i)),
        pl.BlockSpec((NE, STEP), lambda i: (0, i)),
        pl.BlockSpec((2, STEP), lambda i: (0, i)),
        pl.BlockSpec((2, STEP), lambda i: (0, i)),
    ]
    out_shape = (
        jax.ShapeDtypeStruct((NE, N_TOKENS), jnp.float32),
        jax.ShapeDtypeStruct((NE, N_TOKENS), jnp.float32),
        jax.ShapeDtypeStruct((2, N_TOKENS), jnp.int32),
        jax.ShapeDtypeStruct((2, N_TOKENS), jnp.float32),
    )
    logitsT, probsT, idxT, wtsT = pl.pallas_call(
        _router_block,
        grid=(grid,),
        in_specs=in_specs,
        out_specs=out_specs,
        out_shape=out_shape,
    )(*([noise_clock_emb] * NSPLIT), route_weight)
    return (logitsT.T, probsT.T, idxT.T, wtsT.T)


# speedup vs baseline: 1.0744x; 1.0142x over previous
import functools

import jax
import jax.numpy as jnp
from jax import lax
from jax.experimental import pallas as pl

N_TOKENS = 32768
EMB = 768
NE = 8
NSPLIT = 2
Q = 2048
STEP = NSPLIT * Q


def _router_quarter(x, w, j, logitsT_ref, probsT_ref, idxT_ref, wtsT_ref):
    lgT = lax.dot_general(w, x, (((0,), (1,)), ((), ())),
                          preferred_element_type=jnp.float32)
    logitsT_ref[:, pl.ds(j * Q, Q)] = lgT
    mT = jnp.max(lgT, axis=0, keepdims=True)
    eT = jnp.exp(lgT - mT)
    pT = eT / jnp.sum(eT, axis=0, keepdims=True)
    probsT_ref[:, pl.ds(j * Q, Q)] = pT

    iota = lax.broadcasted_iota(jnp.int32, (NE, Q), 0)
    p1 = jnp.max(pT, axis=0, keepdims=True)
    i1 = jnp.min(jnp.where(pT == p1, iota, NE), axis=0, keepdims=True)
    masked = jnp.where(iota == i1, -jnp.inf, pT)
    p2 = jnp.max(masked, axis=0, keepdims=True)
    i2 = jnp.min(jnp.where(masked == p2, iota, NE), axis=0, keepdims=True)
    denom = jnp.maximum(p1 + p2, 1e-8)
    idxT_ref[:, pl.ds(j * Q, Q)] = jnp.concatenate([i1, i2], axis=0)
    wtsT_ref[:, pl.ds(j * Q, Q)] = jnp.concatenate(
        [p1 / denom, p2 / denom], axis=0)


def _router_block(*refs):
    x_refs = refs[:NSPLIT]
    w_ref = refs[NSPLIT]
    logitsT_ref, probsT_ref, idxT_ref, wtsT_ref = refs[NSPLIT + 1:]
    w = w_ref[...]
    for j in range(NSPLIT):
        _router_quarter(x_refs[j][...], w, j,
                        logitsT_ref, probsT_ref, idxT_ref, wtsT_ref)


@jax.jit
def kernel(noise_clock_emb, route_weight):
    grid = N_TOKENS // STEP

    def band(j):
        return pl.BlockSpec((Q, EMB), lambda i, j=j: (NSPLIT * i + j, 0))

    in_specs = [band(j) for j in range(NSPLIT)]
    in_specs.append(pl.BlockSpec((EMB, NE), lambda i: (0, 0)))
    out_specs = [
        pl.BlockSpec((NE, STEP), lambda i: (0, i)),
        pl.BlockSpec((NE, STEP), lambda i: (0, i)),
        pl.BlockSpec((2, STEP), lambda i: (0, i)),
        pl.BlockSpec((2, STEP), lambda i: (0, i)),
    ]
    out_shape = (
        jax.ShapeDtypeStruct((NE, N_TOKENS), jnp.float32),
        jax.ShapeDtypeStruct((NE, N_TOKENS), jnp.float32),
        jax.ShapeDtypeStruct((2, N_TOKENS), jnp.int32),
        jax.ShapeDtypeStruct((2, N_TOKENS), jnp.float32),
    )
    logitsT, probsT, idxT, wtsT = pl.pallas_call(
        _router_block,
        grid=(grid,),
        in_specs=in_specs,
        out_specs=out_specs,
        out_shape=out_shape,
    )(*([noise_clock_emb] * NSPLIT), route_weight)
    return (logitsT.T, probsT.T, idxT.T, wtsT.T)
